# baseline (speedup 1.0000x reference)
import functools

import jax
import jax.numpy as jnp
from jax import lax
from jax.experimental import pallas as pl
from jax.experimental.pallas import tpu as pltpu
from jax.experimental.pallas import tpu_sc as plsc

_NC = 2
_NS = 16
_NW = _NC * _NS
_GS = 16
_L = 16
_KH = 32


def _prep_body(at_ref, tm_ref, idx_ref, table_ref):
    na = at_ref.shape[0]
    best = at_ref[0]
    bidx = jnp.zeros(best.shape, jnp.int32)
    for j in range(1, na):
        v = at_ref[j]
        m = v > best
        best = jnp.where(m, v, best)
        bidx = jnp.where(m, j, bidx)
    idx_ref[...] = bidx
    t = tm_ref[...]
    t = t - jnp.max(t, axis=-1, keepdims=True)
    e = jnp.exp(t)
    table_ref[...] = e / jnp.sum(e, axis=-1, keepdims=True)


def _make_expand_strips(s, c, k, b, na, n_tab):
    n_strip = s * c * k
    pw = n_strip // _NW
    ck = c * k
    mesh = plsc.VectorSubcoreMesh(core_axis_name="c", subcore_axis_name="s")

    @functools.partial(
        pl.kernel,
        out_type=jax.ShapeDtypeStruct((n_strip, k, b), jnp.float32),
        mesh=mesh,
        scratch_types=[
            pltpu.VMEM((b,), jnp.int32),
            pltpu.VMEM((n_tab * k * k,), jnp.float32),
            [pltpu.VMEM((_KH, b), jnp.float32) for _ in range(2)],
            pltpu.SemaphoreType.DMA((2,)),
        ],
        compiler_params=pltpu.CompilerParams(needs_layout_passes=False),
    )
    def expand(table_hbm, idx_hbm, out_hbm, idx_s, table_v, bufs, ssem):
        wid = lax.axis_index("s") * _NC + lax.axis_index("c")
        p0 = wid * pw
        pltpu.sync_copy(table_hbm, table_v)

        def half(p, h, buf, sem, wait_first):
            if wait_first:
                pltpu.make_async_copy(
                    buf, out_hbm.at[p, pl.ds(h * _KH, _KH)], sem).wait()
            k1o = (p % ck) % k * k + h * _KH
            coff = (p % ck) // k * na if c > 1 else 0

            @plsc.parallel_loop(0, b // _L, unroll=2)
            def bchunk(c16):
                b0 = c16 * _L
                row = idx_s[pl.ds(b0, _L)]
                if c > 1:
                    row = row + coff
                pos = row * (k * k) + k1o
                for k2 in range(_KH):
                    val = plsc.load_gather(table_v, [pos + k2])
                    buf[k2, pl.ds(b0, _L)] = val
            pltpu.async_copy(buf, out_hbm.at[p, pl.ds(h * _KH, _KH)], sem)

        def body(t, prev_s):
            p = p0 + t
            s_i = p // ck

            @pl.when(s_i != prev_s)
            def _():
                pltpu.sync_copy(idx_hbm.at[pl.ds(s_i * b, b)], idx_s)

            for h in range(2):
                half(p, h, bufs[h], ssem.at[h], True)
            return s_i

        s_first = p0 // ck
        pltpu.sync_copy(idx_hbm.at[pl.ds(s_first * b, b)], idx_s)
        for h in range(2):
            half(p0, h, bufs[h], ssem.at[h], False)
        lax.fori_loop(1, pw, body, s_first)
        p_last = p0 + pw - 1
        for h in range(2):
            pltpu.make_async_copy(
                bufs[h], out_hbm.at[p_last, pl.ds(h * _KH, _KH)],
                ssem.at[h]).wait()

    return expand


def _make_expand_rows(n_rows, n_tab, k, rpw):
    ngroup = rpw // _GS
    mesh = plsc.VectorSubcoreMesh(core_axis_name="c", subcore_axis_name="s")

    @functools.partial(
        pl.kernel,
        out_type=jax.ShapeDtypeStruct((n_rows, k, k), jnp.float32),
        mesh=mesh,
        scratch_types=[
            pltpu.VMEM((rpw,), jnp.int32),
            pltpu.VMEM((n_tab, k, k), jnp.float32),
            pltpu.SemaphoreType.DMA,
        ],
    )
    def expand(table_hbm, idx_hbm, out_hbm, idx_v, table_v, sem):
        wid = lax.axis_index("s") * _NC + lax.axis_index("c")
        base = wid * rpw
        pltpu.sync_copy(idx_hbm.at[pl.ds(base, rpw)], idx_v)
        pltpu.sync_copy(table_hbm, table_v)

        def body(t, _):
            r0 = t * _GS
            avec = idx_v[pl.ds(r0, _GS)]
            for i in range(_GS):
                a = avec[i]
                pltpu.async_copy(table_v.at[a], out_hbm.at[base + r0 + i],
                                 sem)
            for i in range(_GS):
                pltpu.make_async_copy(
                    table_v.at[0], out_hbm.at[base + r0 + i], sem).wait()
            return 0

        lax.fori_loop(0, ngroup, body, 0)

    return expand


def kernel(action, transition_matrix):
    dim = action.ndim
    if dim == 2:
        action = action[None]
    s, b, na = action.shape
    c, _, k, _ = transition_matrix.shape
    n = s * b

    prep = pl.pallas_call(
        _prep_body,
        out_shape=(
            jax.ShapeDtypeStruct((n,), jnp.int32),
            jax.ShapeDtypeStruct((c, na, k, k), jnp.float32),
        ),
    )
    idx, table = prep(action.reshape(n, na).T, transition_matrix)

    if (s * c * k) % _NW == 0 and k % _KH == 0 and b % _L == 0 and b >= 64:
        table2 = table.reshape(c * na * k * k)
        out = _make_expand_strips(s, c, k, b, na, c * na)(table2, idx)
        prob = out.reshape(s, c, k, k, b).transpose(0, 4, 1, 2, 3)
        if dim == 2:
            prob = prob[0]
        return prob

    rows = idx
    if c > 1:
        rows = (rows[:, None]
                + jnp.arange(c, dtype=jnp.int32)[None, :] * na).reshape(-1)
    nr = n * c
    npad = -(-nr // (_NW * _GS)) * (_NW * _GS)
    if npad != nr:
        rows = jnp.pad(rows, (0, npad - nr))
    rpw = npad // _NW
    table_flat = table.reshape(c * na, k, k)
    out_flat = _make_expand_rows(npad, c * na, k, rpw)(table_flat, rows)
    if npad != nr:
        out_flat = out_flat[:nr]
    prob = out_flat.reshape(s, b, c, k, k)
    if dim == 2:
        prob = prob[0]
    return prob


# speedup vs baseline: 8.0977x; 8.0977x over previous
import functools

import jax
import jax.numpy as jnp
from jax import lax
from jax.experimental import pallas as pl
from jax.experimental.pallas import tpu as pltpu
from jax.experimental.pallas import tpu_sc as plsc

_NC = 2
_NS = 16
_NW = _NC * _NS
_GS = 16
_L = 16
_KH = 32


def _prep_body(at_ref, tm_ref, idx_ref, table_ref):
    na = at_ref.shape[0]
    best = at_ref[0]
    bidx = jnp.zeros(best.shape, jnp.int32)
    for j in range(1, na):
        v = at_ref[j]
        m = v > best
        best = jnp.where(m, v, best)
        bidx = jnp.where(m, j, bidx)
    idx_ref[...] = bidx
    t = tm_ref[...]
    t = t - jnp.max(t, axis=-1, keepdims=True)
    e = jnp.exp(t)
    table_ref[...] = e / jnp.sum(e, axis=-1, keepdims=True)


def _make_expand_strips(s, c, k, b, na, n_tab):
    n_strip = s * c * k
    pw = n_strip // _NW
    ck = c * k
    mesh = plsc.VectorSubcoreMesh(core_axis_name="c", subcore_axis_name="s")

    @functools.partial(
        pl.kernel,
        out_type=jax.ShapeDtypeStruct((n_strip, k, b), jnp.float32),
        mesh=mesh,
        scratch_types=[
            pltpu.VMEM((b,), jnp.int32),
            pltpu.VMEM((n_tab * k * k,), jnp.float32),
            [pltpu.VMEM((_KH, b), jnp.float32) for _ in range(2)],
            pltpu.SemaphoreType.DMA((2,)),
        ],
        compiler_params=pltpu.CompilerParams(needs_layout_passes=False),
    )
    def expand(table_hbm, idx_hbm, out_hbm, idx_s, table_v, bufs, ssem):
        wid = lax.axis_index("s") * _NC + lax.axis_index("c")
        p0 = wid * pw
        pltpu.sync_copy(table_hbm, table_v)

        def half(p, h, buf, sem, wait_first):
            if wait_first:
                pltpu.make_async_copy(
                    buf, out_hbm.at[p, pl.ds(h * _KH, _KH)], sem).wait()
            k1o = (p % ck) % k * k + h * _KH
            coff = (p % ck) // k * na if c > 1 else 0

            @plsc.parallel_loop(0, b // _L, unroll=2)
            def bchunk(c16):
                b0 = c16 * _L
                row = idx_s[pl.ds(b0, _L)]
                if c > 1:
                    row = row + coff
                pos = k1o * n_tab + row
                for k2 in range(_KH):
                    val = plsc.load_gather(table_v, [pos + k2 * n_tab])
                    buf[k2, pl.ds(b0, _L)] = val
            pltpu.async_copy(buf, out_hbm.at[p, pl.ds(h * _KH, _KH)], sem)

        def body(t, prev_s):
            p = p0 + t
            s_i = p // ck

            @pl.when(s_i != prev_s)
            def _():
                pltpu.sync_copy(idx_hbm.at[pl.ds(s_i * b, b)], idx_s)

            for h in range(2):
                half(p, h, bufs[h], ssem.at[h], True)
            return s_i

        s_first = p0 // ck
        pltpu.sync_copy(idx_hbm.at[pl.ds(s_first * b, b)], idx_s)
        for h in range(2):
            half(p0, h, bufs[h], ssem.at[h], False)
        lax.fori_loop(1, pw, body, s_first)
        p_last = p0 + pw - 1
        for h in range(2):
            pltpu.make_async_copy(
                bufs[h], out_hbm.at[p_last, pl.ds(h * _KH, _KH)],
                ssem.at[h]).wait()

    return expand


def _make_expand_rows(n_rows, n_tab, k, rpw):
    ngroup = rpw // _GS
    mesh = plsc.VectorSubcoreMesh(core_axis_name="c", subcore_axis_name="s")

    @functools.partial(
        pl.kernel,
        out_type=jax.ShapeDtypeStruct((n_rows, k, k), jnp.float32),
        mesh=mesh,
        scratch_types=[
            pltpu.VMEM((rpw,), jnp.int32),
            pltpu.VMEM((n_tab, k, k), jnp.float32),
            pltpu.SemaphoreType.DMA,
        ],
    )
    def expand(table_hbm, idx_hbm, out_hbm, idx_v, table_v, sem):
        wid = lax.axis_index("s") * _NC + lax.axis_index("c")
        base = wid * rpw
        pltpu.sync_copy(idx_hbm.at[pl.ds(base, rpw)], idx_v)
        pltpu.sync_copy(table_hbm, table_v)

        def body(t, _):
            r0 = t * _GS
            avec = idx_v[pl.ds(r0, _GS)]
            for i in range(_GS):
                a = avec[i]
                pltpu.async_copy(table_v.at[a], out_hbm.at[base + r0 + i],
                                 sem)
            for i in range(_GS):
                pltpu.make_async_copy(
                    table_v.at[0], out_hbm.at[base + r0 + i], sem).wait()
            return 0

        lax.fori_loop(0, ngroup, body, 0)

    return expand


def kernel(action, transition_matrix):
    dim = action.ndim
    if dim == 2:
        action = action[None]
    s, b, na = action.shape
    c, _, k, _ = transition_matrix.shape
    n = s * b

    prep = pl.pallas_call(
        _prep_body,
        out_shape=(
            jax.ShapeDtypeStruct((n,), jnp.int32),
            jax.ShapeDtypeStruct((c, na, k, k), jnp.float32),
        ),
    )
    idx, table = prep(action.reshape(n, na).T, transition_matrix)

    if (s * c * k) % _NW == 0 and k % _KH == 0 and b % _L == 0 and b >= 64:
        table2 = jnp.transpose(table, (2, 3, 0, 1)).reshape(-1)
        out = _make_expand_strips(s, c, k, b, na, c * na)(table2, idx)
        prob = out.reshape(s, c, k, k, b).transpose(0, 4, 1, 2, 3)
        if dim == 2:
            prob = prob[0]
        return prob

    rows = idx
    if c > 1:
        rows = (rows[:, None]
                + jnp.arange(c, dtype=jnp.int32)[None, :] * na).reshape(-1)
    nr = n * c
    npad = -(-nr // (_NW * _GS)) * (_NW * _GS)
    if npad != nr:
        rows = jnp.pad(rows, (0, npad - nr))
    rpw = npad // _NW
    table_flat = table.reshape(c * na, k, k)
    out_flat = _make_expand_rows(npad, c * na, k, rpw)(table_flat, rows)
    if npad != nr:
        out_flat = out_flat[:nr]
    prob = out_flat.reshape(s, b, c, k, k)
    if dim == 2:
        prob = prob[0]
    return prob
